# baseline (speedup 1.0000x reference)
import functools

import jax
import jax.numpy as jnp
from jax import lax
from jax.experimental import pallas as pl
from jax.experimental.pallas import tpu as pltpu
from jax.experimental.pallas import tpu_sc as plsc

K = 8
CAPACITY_FACTOR = 1.25


_SC_CORES = 2
_SC_SUBCORES = 16
_NW = _SC_CORES * _SC_SUBCORES


def _sc_scatter_pairs(vals, addr, n_rows):
    n, w = vals.shape
    chunk = 128
    per_w = n // _NW
    k_chunks = per_w // chunk
    addr3 = addr.reshape(_NW, k_chunks, chunk)
    mesh = plsc.VectorSubcoreMesh(core_axis_name="c", subcore_axis_name="s")

    @functools.partial(
        pl.kernel, mesh=mesh,
        out_type=jax.ShapeDtypeStruct((n_rows, w), jnp.float32),
        scratch_types=[
            pltpu.VMEM((k_chunks, chunk), jnp.int32),
            pltpu.VMEM((chunk, w), jnp.float32),
            pltpu.SemaphoreType.DMA,
        ],
    )
    def k(vals_hbm, addr_hbm, table_hbm, idx_v, vals_v, sem):
        wid = lax.axis_index("s") * _SC_CORES + lax.axis_index("c")
        base = wid * per_w
        pltpu.sync_copy(addr_hbm.at[wid], idx_v)

        @pl.loop(0, k_chunks)
        def _(ck):
            pltpu.sync_copy(vals_hbm.at[pl.ds(base + ck * chunk, chunk)],
                            vals_v)
            pltpu.sync_copy(vals_v, table_hbm.at[idx_v.at[ck]])

    return k(vals, addr3)


def _sc_gather_rows(x2, tmap):
    t_rows, c_dim = x2.shape
    n = tmap.shape[0]
    chunk = 32
    per_w = n // _NW
    k_chunks = per_w // chunk
    mesh = plsc.VectorSubcoreMesh(core_axis_name="c", subcore_axis_name="s")

    @functools.partial(
        pl.kernel, mesh=mesh,
        out_type=jax.ShapeDtypeStruct((n, c_dim), jnp.float32),
        scratch_types=[
            pltpu.VMEM((chunk,), jnp.int32),
            pltpu.VMEM((chunk, c_dim), jnp.float32),
            pltpu.SemaphoreType.DMA,
        ],
    )
    def k(x_hbm, idx_hbm, out_hbm, idx_v, rows_v, sem):
        wid = lax.axis_index("s") * _SC_CORES + lax.axis_index("c")
        base = wid * per_w

        @pl.loop(0, k_chunks)
        def _(ck):
            off = base + ck * chunk
            pltpu.sync_copy(idx_hbm.at[pl.ds(off, chunk)], idx_v)
            pltpu.async_copy(x_hbm.at[idx_v], rows_v, sem).wait()
            pltpu.sync_copy(rows_v, out_hbm.at[pl.ds(off, chunk)])

    return k(x2, tmap)


def _ffn_combine_kernel(counts_ref, tmap_ref, buf_ref, w1a_ref, w1b_ref,
                        w2a_ref, w2b_ref, w2c_ref, w2d_ref, b1_ref, b2_ref,
                        p_ref, out_ref, h4_ref, yacc_ref, *, n_ff, r, fb):
    e = pl.program_id(0)

    @pl.when(e == 0)
    def _():
        out_ref[...] = jnp.zeros_like(out_ref)

    xb = buf_ref[0].astype(jnp.bfloat16)
    ch = xb.shape[1] // 2
    xa = xb[:, :ch]
    xc = xb[:, ch:]
    for j in range(n_ff):
        sl = slice(j * fb, (j + 1) * fb)
        w1j_a = w1a_ref[0, 0][:, sl].astype(jnp.bfloat16)
        w1j_b = w1b_ref[0, 0][:, sl].astype(jnp.bfloat16)
        hj = (jnp.dot(xa, w1j_a, preferred_element_type=jnp.float32)
              + jnp.dot(xc, w1j_b, preferred_element_type=jnp.float32))
        hj = jnp.maximum(hj + b1_ref[0][:, sl], 0.0)
        h4_ref[j] = hj.astype(jnp.bfloat16)

    w2refs = [w2a_ref, w2b_ref, w2c_ref, w2d_ref]
    for j in range(n_ff):
        w2j = w2refs[j][0, 0].astype(jnp.bfloat16)
        y = jnp.dot(h4_ref[j], w2j, preferred_element_type=jnp.float32)
        if j == 0:
            yacc_ref[...] = y
        else:
            yacc_ref[...] += y

    cnt = jnp.minimum(counts_ref[e], r)
    sidx = jax.lax.broadcasted_iota(jnp.int32, (r, 1), 0)
    w = jnp.where(sidx < cnt, p_ref[0], 0.0)
    yacc_ref[...] = (yacc_ref[...] + b2_ref[0]) * w

    def body(i, _):
        t = tmap_ref[e * r + i]
        row = yacc_ref[pl.ds(i, 1), :]
        out_ref[pl.ds(t, 1), :] = out_ref[pl.ds(t, 1), :] + row
        return 0

    jax.lax.fori_loop(0, r, body, 0, unroll=4)


def _run_ffn_combine(counts, tmap, buf, W1, b1, W2, b2, p_col, n_tokens,
                     interpret=False):
    E, R, C = buf.shape
    D_FF = W1.shape[2]
    n_ff = 4 if D_FF % 4 == 0 else 1
    fb = D_FF // n_ff
    W1r = W1.reshape(E, 2, C // 2, D_FF)
    W2r = W2.reshape(E, n_ff, fb, C)

    grid_spec = pltpu.PrefetchScalarGridSpec(
        num_scalar_prefetch=2,
        grid=(E,),
        in_specs=[
            pl.BlockSpec((1, R, C), lambda e, *_: (e, 0, 0)),
            pl.BlockSpec((1, 1, C // 2, D_FF), lambda e, *_: (e, 0, 0, 0)),
            pl.BlockSpec((1, 1, C // 2, D_FF), lambda e, *_: (e, 1, 0, 0)),
            pl.BlockSpec((1, 1, fb, C), lambda e, *_: (e, 0, 0, 0)),
            pl.BlockSpec((1, 1, fb, C), lambda e, *_: (e, 1, 0, 0)),
            pl.BlockSpec((1, 1, fb, C), lambda e, *_: (e, 2, 0, 0)),
            pl.BlockSpec((1, 1, fb, C), lambda e, *_: (e, 3, 0, 0)),
            pl.BlockSpec((1, 1, D_FF), lambda e, *_: (e, 0, 0)),
            pl.BlockSpec((1, 1, C), lambda e, *_: (e, 0, 0)),
            pl.BlockSpec((1, R, 1), lambda e, *_: (e, 0, 0)),
        ],
        out_specs=pl.BlockSpec((n_tokens, C), lambda e, *_: (0, 0)),
        scratch_shapes=[pltpu.VMEM((n_ff, R, fb), jnp.bfloat16),
                        pltpu.VMEM((R, C), jnp.float32)],
    )
    kernel = pl.pallas_call(
        functools.partial(_ffn_combine_kernel, n_ff=n_ff, r=R, fb=fb),
        grid_spec=grid_spec,
        out_shape=jax.ShapeDtypeStruct((n_tokens, C), jnp.float32),
        compiler_params=pltpu.CompilerParams(
            dimension_semantics=("arbitrary",),
            vmem_limit_bytes=128 * 1024 * 1024,
        ),
        interpret=interpret,
    )
    b1r = b1.reshape(E, 1, D_FF)
    b2r = b2.reshape(E, 1, C)
    return kernel(counts, tmap, buf, W1r, W1r, W2r, W2r, W2r, W2r, b1r, b2r, p_col)


def _router_kernel(x_ref, wr_ref, br_ref, addr_ref, pval_ref, counts_ref,
                   carry_ref, ltri_ref, *, tb, e_num, cap, n_blocks):
    i = pl.program_id(0)

    @pl.when(i == 0)
    def _():
        carry_ref[...] = jnp.zeros_like(carry_ref)
        r_iota = jax.lax.broadcasted_iota(jnp.int32, (tb, tb), 0)
        c_iota = jax.lax.broadcasted_iota(jnp.int32, (tb, tb), 1)
        ltri_ref[...] = (r_iota > c_iota).astype(jnp.bfloat16)

    xb = x_ref[...]
    logits = jnp.dot(xb, wr_ref[...], preferred_element_type=jnp.float32)
    logits = logits + br_ref[...]
    m = jnp.max(logits, axis=1, keepdims=True)
    el = jnp.exp(logits - m)
    z = jnp.sum(el, axis=1, keepdims=True)
    iota_e = jax.lax.broadcasted_iota(jnp.int32, (tb, e_num), 1)

    cur = logits
    ohsum = jnp.zeros((tb, e_num), jnp.float32)
    eks, pks = [], []
    for _ in range(K):
        mx = jnp.max(cur, axis=1, keepdims=True)
        idx = jnp.min(jnp.where(cur == mx, iota_e, e_num), axis=1,
                      keepdims=True)
        msk = iota_e == idx
        pks.append(jnp.sum(jnp.where(msk, el, 0.0), axis=1, keepdims=True) / z)
        ohsum = ohsum + msk.astype(jnp.float32)
        cur = jnp.where(msk, -jnp.inf, cur)
        eks.append(idx)

    exc = jnp.dot(ltri_ref[...], ohsum.astype(jnp.bfloat16),
                  preferred_element_type=jnp.float32)
    exc = exc + carry_ref[...]

    poss = []
    for k in range(K):
        v = jnp.sum(jnp.where(iota_e == eks[k], exc, 0.0), axis=1,
                    keepdims=True)
        poss.append(v)
    pos = jnp.concatenate(poss, axis=1).astype(jnp.int32) + 1
    ek = jnp.concatenate(eks, axis=1)
    pk = jnp.concatenate(pks, axis=1)
    keep = pos <= cap
    addr_ref[...] = jnp.where(keep, ek * cap + (pos - 1), e_num * cap)
    pval_ref[...] = jnp.where(keep, pk, 0.0)
    carry_ref[...] += jnp.sum(ohsum, axis=0, keepdims=True)

    @pl.when(i == n_blocks - 1)
    def _():
        counts_ref[...] = carry_ref[...].astype(jnp.int32)


def _run_router(x2, W_r, b_r, cap, interpret=False):
    T, C = x2.shape
    E = W_r.shape[1]
    tb = 512 if T % 512 == 0 else T
    n_blocks = T // tb
    out_shapes = (
        jax.ShapeDtypeStruct((T, K), jnp.int32),
        jax.ShapeDtypeStruct((T, K), jnp.float32),
        jax.ShapeDtypeStruct((1, E), jnp.int32),
    )
    return pl.pallas_call(
        functools.partial(_router_kernel, tb=tb, e_num=E, cap=cap,
                          n_blocks=n_blocks),
        grid=(n_blocks,),
        in_specs=[
            pl.BlockSpec((tb, C), lambda i: (i, 0)),
            pl.BlockSpec((C, E), lambda i: (0, 0)),
            pl.BlockSpec((1, E), lambda i: (0, 0)),
        ],
        out_specs=(
            pl.BlockSpec((tb, K), lambda i: (i, 0)),
            pl.BlockSpec((tb, K), lambda i: (i, 0)),
            pl.BlockSpec((1, E), lambda i: (0, 0)),
        ),
        out_shape=out_shapes,
        scratch_shapes=[pltpu.VMEM((1, E), jnp.float32),
                        pltpu.VMEM((tb, tb), jnp.bfloat16)],
        compiler_params=pltpu.CompilerParams(
            dimension_semantics=("arbitrary",),
        ),
        interpret=interpret,
    )(x2, W_r, b_r.reshape(1, E))


def kernel(x, W_r, b_r, W1, b1, W2, b2):
    B, T, C = x.shape
    E = W_r.shape[1]
    cap = max(1, int(T / E * CAPACITY_FACTOR))
    R = cap

    x2 = x.reshape(T, C)
    addr, pval, counts2 = _run_router(x2, W_r, b_r, cap)
    n = T * K
    tvals = jnp.broadcast_to(
        jnp.arange(T, dtype=jnp.float32)[:, None], (T, K))
    vals = jnp.stack([tvals.reshape(n), pval.reshape(n)], axis=-1)

    n_rows = E * R + 8
    table = jnp.zeros((n_rows, 2), jnp.float32).at[addr.reshape(n)].set(
        vals, mode='drop')
    tmap = table[:E * R, 0].astype(jnp.int32)
    p_col = table[:E * R, 1].reshape(E, R, 1)
    counts = counts2.reshape(E)

    buf = _sc_gather_rows(x2, tmap).reshape(E, R, C)

    out = _run_ffn_combine(counts, tmap, buf, W1, b1, W2, b2, p_col, T)
    return out.reshape(B, T, C)


# speedup vs baseline: 1.0117x; 1.0117x over previous
import functools

import jax
import jax.numpy as jnp
from jax import lax
from jax.experimental import pallas as pl
from jax.experimental.pallas import tpu as pltpu
from jax.experimental.pallas import tpu_sc as plsc

K = 8
CAPACITY_FACTOR = 1.25


_SC_CORES = 2
_SC_SUBCORES = 16
_NW = _SC_CORES * _SC_SUBCORES


def _sc_gather_rows(x2, tmap):
    t_rows, c_dim = x2.shape
    n = tmap.shape[0]
    chunk = 32
    per_w = n // _NW
    k_chunks = per_w // chunk
    mesh = plsc.VectorSubcoreMesh(core_axis_name="c", subcore_axis_name="s")

    @functools.partial(
        pl.kernel, mesh=mesh,
        out_type=jax.ShapeDtypeStruct((n, c_dim), jnp.float32),
        scratch_types=[
            pltpu.VMEM((chunk,), jnp.int32),
            pltpu.VMEM((chunk, c_dim), jnp.float32),
            pltpu.SemaphoreType.DMA,
        ],
    )
    def k(x_hbm, idx_hbm, out_hbm, idx_v, rows_v, sem):
        wid = lax.axis_index("s") * _SC_CORES + lax.axis_index("c")
        base = wid * per_w

        @pl.loop(0, k_chunks)
        def _(ck):
            off = base + ck * chunk
            pltpu.sync_copy(idx_hbm.at[pl.ds(off, chunk)], idx_v)
            pltpu.async_copy(x_hbm.at[idx_v], rows_v, sem).wait()
            pltpu.sync_copy(rows_v, out_hbm.at[pl.ds(off, chunk)])

    return k(x2, tmap)


def _ffn_combine_kernel(counts_ref, tmap_ref, buf_ref, w1a_ref, w1b_ref,
                        w2a_ref, w2b_ref, w2c_ref, w2d_ref, b1_ref, b2_ref,
                        p_ref, out_ref, h4_ref, yacc_ref, *, n_ff, r, fb):
    e = pl.program_id(0)

    @pl.when(e == 0)
    def _():
        out_ref[...] = jnp.zeros_like(out_ref)

    xb = buf_ref[0].astype(jnp.bfloat16)
    ch = xb.shape[1] // 2
    xa = xb[:, :ch]
    xc = xb[:, ch:]
    for j in range(n_ff):
        sl = slice(j * fb, (j + 1) * fb)
        w1j_a = w1a_ref[0, 0][:, sl].astype(jnp.bfloat16)
        w1j_b = w1b_ref[0, 0][:, sl].astype(jnp.bfloat16)
        hj = (jnp.dot(xa, w1j_a, preferred_element_type=jnp.float32)
              + jnp.dot(xc, w1j_b, preferred_element_type=jnp.float32))
        hj = jnp.maximum(hj + b1_ref[0][:, sl], 0.0)
        h4_ref[j] = hj.astype(jnp.bfloat16)

    w2refs = [w2a_ref, w2b_ref, w2c_ref, w2d_ref]
    for j in range(n_ff):
        w2j = w2refs[j][0, 0].astype(jnp.bfloat16)
        y = jnp.dot(h4_ref[j], w2j, preferred_element_type=jnp.float32)
        if j == 0:
            yacc_ref[...] = y
        else:
            yacc_ref[...] += y

    cnt = jnp.minimum(counts_ref[e], r)
    sidx = jax.lax.broadcasted_iota(jnp.int32, (r, 1), 0)
    w = jnp.where(sidx < cnt, p_ref[0], 0.0)
    yacc_ref[...] = (yacc_ref[...] + b2_ref[0]) * w

    def body(i, _):
        t = tmap_ref[e * r + i]
        row = yacc_ref[pl.ds(i, 1), :]
        out_ref[pl.ds(t, 1), :] = out_ref[pl.ds(t, 1), :] + row
        return 0

    jax.lax.fori_loop(0, r, body, 0, unroll=4)


def _run_ffn_combine(counts, tmap, buf, W1, b1, W2, b2, p_col, n_tokens,
                     interpret=False):
    E, R, C = buf.shape
    D_FF = W1.shape[2]
    n_ff = 4 if D_FF % 4 == 0 else 1
    fb = D_FF // n_ff
    W1r = W1.reshape(E, 2, C // 2, D_FF)
    W2r = W2.reshape(E, n_ff, fb, C)

    grid_spec = pltpu.PrefetchScalarGridSpec(
        num_scalar_prefetch=2,
        grid=(E,),
        in_specs=[
            pl.BlockSpec((1, R, C), lambda e, *_: (e, 0, 0)),
            pl.BlockSpec((1, 1, C // 2, D_FF), lambda e, *_: (e, 0, 0, 0)),
            pl.BlockSpec((1, 1, C // 2, D_FF), lambda e, *_: (e, 1, 0, 0)),
            pl.BlockSpec((1, 1, fb, C), lambda e, *_: (e, 0, 0, 0)),
            pl.BlockSpec((1, 1, fb, C), lambda e, *_: (e, 1, 0, 0)),
            pl.BlockSpec((1, 1, fb, C), lambda e, *_: (e, 2, 0, 0)),
            pl.BlockSpec((1, 1, fb, C), lambda e, *_: (e, 3, 0, 0)),
            pl.BlockSpec((1, 1, D_FF), lambda e, *_: (e, 0, 0)),
            pl.BlockSpec((1, 1, C), lambda e, *_: (e, 0, 0)),
            pl.BlockSpec((1, R, 1), lambda e, *_: (e, 0, 0)),
        ],
        out_specs=pl.BlockSpec((n_tokens, C), lambda e, *_: (0, 0)),
        scratch_shapes=[pltpu.VMEM((n_ff, R, fb), jnp.bfloat16),
                        pltpu.VMEM((R, C), jnp.float32)],
    )
    kernel = pl.pallas_call(
        functools.partial(_ffn_combine_kernel, n_ff=n_ff, r=R, fb=fb),
        grid_spec=grid_spec,
        out_shape=jax.ShapeDtypeStruct((n_tokens, C), jnp.float32),
        compiler_params=pltpu.CompilerParams(
            dimension_semantics=("arbitrary",),
            vmem_limit_bytes=128 * 1024 * 1024,
        ),
        interpret=interpret,
    )
    b1r = b1.reshape(E, 1, D_FF)
    b2r = b2.reshape(E, 1, C)
    return kernel(counts, tmap, buf, W1r, W1r, W2r, W2r, W2r, W2r, b1r, b2r, p_col)


def _router_kernel(x_ref, wr_ref, br_ref, addr_ref, pval_ref, counts_ref,
                   carry_ref, *, tb, e_num, cap, n_blocks):
    i = pl.program_id(0)

    @pl.when(i == 0)
    def _():
        carry_ref[...] = jnp.zeros_like(carry_ref)

    xb = x_ref[...]
    logits = jnp.dot(xb, wr_ref[...], preferred_element_type=jnp.float32)
    logits = logits + br_ref[...]
    m = jnp.max(logits, axis=1, keepdims=True)
    el = jnp.exp(logits - m)
    z = jnp.sum(el, axis=1, keepdims=True)
    iota_e = jax.lax.broadcasted_iota(jnp.int32, (tb, e_num), 1)

    cur = logits
    ohsum = jnp.zeros((tb, e_num), jnp.float32)
    eks, pks = [], []
    for _ in range(K):
        mx = jnp.max(cur, axis=1, keepdims=True)
        idx = jnp.min(jnp.where(cur == mx, iota_e, e_num), axis=1,
                      keepdims=True)
        msk = iota_e == idx
        pks.append(jnp.sum(jnp.where(msk, el, 0.0), axis=1, keepdims=True) / z)
        ohsum = ohsum + msk.astype(jnp.float32)
        cur = jnp.where(msk, -jnp.inf, cur)
        eks.append(idx)

    r_iota = jax.lax.broadcasted_iota(jnp.int32, (tb, tb), 0)
    c_iota = jax.lax.broadcasted_iota(jnp.int32, (tb, tb), 1)
    ltri = (r_iota > c_iota).astype(jnp.float32)
    exc = jnp.dot(ltri, ohsum, preferred_element_type=jnp.float32)
    exc = exc + carry_ref[...]

    poss = []
    for k in range(K):
        v = jnp.sum(jnp.where(iota_e == eks[k], exc, 0.0), axis=1,
                    keepdims=True)
        poss.append(v)
    pos = jnp.concatenate(poss, axis=1).astype(jnp.int32) + 1
    ek = jnp.concatenate(eks, axis=1)
    pk = jnp.concatenate(pks, axis=1)
    keep = pos <= cap
    addr_ref[...] = jnp.where(keep, ek * cap + (pos - 1), e_num * cap)
    pval_ref[...] = jnp.where(keep, pk, 0.0)
    carry_ref[...] += jnp.sum(ohsum, axis=0, keepdims=True)

    @pl.when(i == n_blocks - 1)
    def _():
        counts_ref[...] = carry_ref[...].astype(jnp.int32)


def _run_router(x2, W_r, b_r, cap, interpret=False):
    T, C = x2.shape
    E = W_r.shape[1]
    tb = 512 if T % 512 == 0 else T
    n_blocks = T // tb
    out_shapes = (
        jax.ShapeDtypeStruct((T, K), jnp.int32),
        jax.ShapeDtypeStruct((T, K), jnp.float32),
        jax.ShapeDtypeStruct((1, E), jnp.int32),
    )
    return pl.pallas_call(
        functools.partial(_router_kernel, tb=tb, e_num=E, cap=cap,
                          n_blocks=n_blocks),
        grid=(n_blocks,),
        in_specs=[
            pl.BlockSpec((tb, C), lambda i: (i, 0)),
            pl.BlockSpec((C, E), lambda i: (0, 0)),
            pl.BlockSpec((1, E), lambda i: (0, 0)),
        ],
        out_specs=(
            pl.BlockSpec((tb, K), lambda i: (i, 0)),
            pl.BlockSpec((tb, K), lambda i: (i, 0)),
            pl.BlockSpec((1, E), lambda i: (0, 0)),
        ),
        out_shape=out_shapes,
        scratch_shapes=[pltpu.VMEM((1, E), jnp.float32)],
        compiler_params=pltpu.CompilerParams(
            dimension_semantics=("arbitrary",),
        ),
        interpret=interpret,
    )(x2, W_r, b_r.reshape(1, E))


def kernel(x, W_r, b_r, W1, b1, W2, b2):
    B, T, C = x.shape
    E = W_r.shape[1]
    cap = max(1, int(T / E * CAPACITY_FACTOR))
    R = cap

    x2 = x.reshape(T, C)
    addr, pval, counts2 = _run_router(x2, W_r, b_r, cap)
    n = T * K
    tvals = jnp.broadcast_to(
        jnp.arange(T, dtype=jnp.float32)[:, None], (T, K))
    vals = jnp.stack([tvals.reshape(n), pval.reshape(n)], axis=-1)

    n_rows = E * R + 8
    table = jnp.zeros((n_rows, 2), jnp.float32).at[addr.reshape(n)].set(
        vals, mode='drop')
    tmap = table[:E * R, 0].astype(jnp.int32)
    p_col = table[:E * R, 1].reshape(E, R, 1)
    counts = counts2.reshape(E)

    buf = _sc_gather_rows(x2, tmap).reshape(E, R, C)

    out = _run_ffn_combine(counts, tmap, buf, W1, b1, W2, b2, p_col, T)
    return out.reshape(B, T, C)
